# baseline (speedup 1.0000x reference)
import functools

import jax
import jax.numpy as jnp
from jax import lax
from jax.experimental import pallas as pl
from jax.experimental.pallas import tpu as pltpu
from jax.experimental.pallas import tpu_sc as plsc

MAX_LEN = 200
EMBED_DIM = 128
BATCH = 1024
NUM_CORES = 2
NUM_SUBCORES = 16
NUM_WORKERS = NUM_CORES * NUM_SUBCORES
CHUNK = 128
IDX_PER_WORKER = BATCH * MAX_LEN // NUM_WORKERS
NUM_CHUNKS = IDX_PER_WORKER // CHUNK
NBUF = 7
DEPTH = 4
LANES = 16


def _sc_embed(idx, table, pe2d):
    mesh = plsc.VectorSubcoreMesh(core_axis_name="c", subcore_axis_name="s")

    @functools.partial(
        pl.kernel,
        mesh=mesh,
        out_type=jax.ShapeDtypeStruct((BATCH * MAX_LEN, EMBED_DIM), jnp.float32),
        scratch_types=[
            pltpu.VMEM((NUM_CHUNKS, CHUNK), jnp.int32),
            *([pltpu.VMEM((CHUNK, EMBED_DIM), jnp.float32)] * NBUF),
            pltpu.VMEM_SHARED((MAX_LEN, EMBED_DIM), jnp.float32),
            *([pltpu.SemaphoreType.DMA] * (2 * NBUF)),
        ],
    )
    def k(idx_hbm, table_hbm, pe_hbm, out_hbm, idx_v, *rest):
        bufs = rest[:NBUF]
        pe_sh = rest[NBUF]
        gsems = rest[NBUF + 1:2 * NBUF + 1]
        fsems = rest[2 * NBUF + 1:3 * NBUF + 1]
        sid = lax.axis_index("s")
        wid = sid * NUM_CORES + lax.axis_index("c")
        base = wid * IDX_PER_WORKER
        pltpu.sync_copy(idx_hbm.at[wid], idx_v)

        @pl.when(sid == 0)
        def _():
            pltpu.sync_copy(pe_hbm, pe_sh)

        plsc.subcore_barrier()

        def prep(gg):
            s = gg % NBUF
            p0 = (gg * CHUNK) % MAX_LEN
            n1 = min(MAX_LEN - p0, CHUNK)
            pltpu.sync_copy(pe_sh.at[pl.ds(p0, n1)], bufs[s].at[pl.ds(0, n1)])
            if n1 < CHUNK:
                pltpu.sync_copy(pe_sh.at[pl.ds(0, CHUNK - n1)],
                                bufs[s].at[pl.ds(n1, CHUNK - n1)])
            pltpu.async_copy(table_hbm.at[idx_v.at[gg]], bufs[s], gsems[s],
                             add=True)

        def finish(gg):
            s = gg % NBUF
            pltpu.make_async_copy(table_hbm.at[idx_v.at[gg]], bufs[s],
                                  gsems[s]).wait()
            pltpu.async_copy(bufs[s], out_hbm.at[pl.ds(base + gg * CHUNK, CHUNK)],
                             fsems[s])

        def wait_flush(gg):
            s = gg % NBUF
            pltpu.make_async_copy(bufs[s],
                                  out_hbm.at[pl.ds(base + gg * CHUNK, CHUNK)],
                                  fsems[s]).wait()

        for gg in range(DEPTH):
            prep(gg)
        for gg in range(NUM_CHUNKS):
            nxt = gg + DEPTH
            if nxt < NUM_CHUNKS:
                if nxt - NBUF >= 0:
                    wait_flush(nxt - NBUF)
                prep(nxt)
            finish(gg)
        for gg in range(max(0, NUM_CHUNKS - NBUF), NUM_CHUNKS):
            wait_flush(gg)

    return k(idx, table, pe2d)


def kernel(x, embed_weight, pe):
    idx = x.astype(jnp.int32).reshape(NUM_WORKERS, NUM_CHUNKS, CHUNK)
    out = _sc_embed(idx, embed_weight, pe.reshape(MAX_LEN, EMBED_DIM))
    return out.reshape(BATCH, MAX_LEN, EMBED_DIM)
